# Initial kernel scaffold; baseline (speedup 1.0000x reference)
import jax
import jax.numpy as jnp
from jax.experimental import pallas as pl


def kernel(mu, Sigma):
    raise NotImplementedError("write your pallas kernel here")


# speedup vs baseline: 1369.5762x; 1369.5762x over previous
import jax
import jax.numpy as jnp
from jax.experimental import pallas as pl

_N = 4096
_BR = 256


def _diag_body(sig_blk, d_out):
    r = jax.lax.broadcasted_iota(jnp.int32, (_BR, _BR), 0)
    c = jax.lax.broadcasted_iota(jnp.int32, (_BR, _BR), 1)
    d_out[...] = jnp.sum(
        jnp.where(r == c, sig_blk[...], 0.0), axis=1, keepdims=True
    )


def _main_body(sig_blk, mu_row, mu_col, d_row, d_col, mud_out, sd_out):
    i = pl.program_id(0)
    s = sig_blk[...]
    sd_full = (d_col[...] + d_row[...]) - 2.0 * s
    mud_full = mu_col[...] - mu_row[...]
    j_col = i * _BR + jax.lax.broadcasted_iota(jnp.int32, (_BR, _N - 1), 0)
    m_col = jax.lax.broadcasted_iota(jnp.int32, (_BR, _N - 1), 1)
    keep_lo = m_col < j_col
    sd_c = jnp.where(keep_lo, sd_full[:, : _N - 1], sd_full[:, 1:])
    mud_c = jnp.where(keep_lo, mud_full[:, : _N - 1], mud_full[:, 1:])
    sd_out[...] = jnp.sqrt(sd_c)
    mud_out[...] = mud_c


def kernel(mu, Sigma):
    n = _N
    d_col = pl.pallas_call(
        _diag_body,
        grid=(n // _BR,),
        in_specs=[pl.BlockSpec((_BR, _BR), lambda i: (i, i))],
        out_specs=pl.BlockSpec((_BR, 1), lambda i: (i, 0)),
        out_shape=jax.ShapeDtypeStruct((n, 1), jnp.float32),
    )(Sigma)

    d_row = d_col.reshape(1, n)
    mu_col = mu.reshape(n, 1)
    mu_row = mu.reshape(1, n)

    mud2, sd2 = pl.pallas_call(
        _main_body,
        grid=(n // _BR,),
        in_specs=[
            pl.BlockSpec((_BR, n), lambda i: (i, 0)),
            pl.BlockSpec((1, n), lambda i: (0, 0)),
            pl.BlockSpec((_BR, 1), lambda i: (i, 0)),
            pl.BlockSpec((1, n), lambda i: (0, 0)),
            pl.BlockSpec((_BR, 1), lambda i: (i, 0)),
        ],
        out_specs=[
            pl.BlockSpec((_BR, n - 1), lambda i: (i, 0)),
            pl.BlockSpec((_BR, n - 1), lambda i: (i, 0)),
        ],
        out_shape=[
            jax.ShapeDtypeStruct((n, n - 1), jnp.float32),
            jax.ShapeDtypeStruct((n, n - 1), jnp.float32),
        ],
    )(Sigma, mu_row, mu_col, d_row, d_col)

    return mud2.reshape(-1), sd2.reshape(-1)
